# Initial kernel scaffold; baseline (speedup 1.0000x reference)
import jax
import jax.numpy as jnp
from jax.experimental import pallas as pl


def kernel(inputs, embeddings):
    raise NotImplementedError("write your pallas kernel here")


# speedup vs baseline: 1.4247x; 1.4247x over previous
import functools

import jax
import jax.numpy as jnp
from jax.experimental import pallas as pl
from jax.experimental.pallas import tpu as pltpu

D = 256
K = 8192
B = 8
HW = 1024
N = B * HW
BLK = 256
GRID = N // BLK
COMMIT = 0.25


def _vq_body(x_ref, e_ref, e2_ref, dist_ref, enc_ref, q_ref, idx_ref,
             loss_ref, ppl_ref, idx_all, loss_acc):
    i = pl.program_id(0)

    xb = x_ref[...]
    mm = jax.lax.dot_general(
        xb, e_ref[...], (((1,), (0,)), ((), ())),
        preferred_element_type=jnp.float32)
    x2 = jnp.sum(xb * xb, axis=1, keepdims=True)
    dist = (x2 - 2.0 * mm) + e2_ref[...]
    dist_ref[...] = dist

    min_d = jnp.min(dist, axis=1, keepdims=True)
    iota = jax.lax.broadcasted_iota(jnp.int32, (BLK, K), 1)
    sel = jnp.where(dist == min_d, iota, jnp.int32(2147483647))
    idx = jnp.min(sel, axis=1).astype(jnp.int32)
    idx_ref[...] = idx[None, None, :]
    idx_all[i, :] = idx

    enc = (iota == idx[:, None]).astype(jnp.float32)
    enc_ref[...] = enc

    q_ref[...] = jax.lax.dot_general(
        enc, e_ref[...], (((1,), (1,)), ((), ())),
        preferred_element_type=jnp.float32,
        precision=jax.lax.Precision.HIGHEST)

    @pl.when(i == 0)
    def _():
        loss_acc[0, 0] = 0.0

    loss_acc[0, 0] += jnp.sum(min_d)

    @pl.when(i == GRID - 1)
    def _():
        loss_ref[...] = jnp.full(
            (1, 1), loss_acc[0, 0] * ((1.0 + COMMIT) / (N * D)),
            dtype=jnp.float32)
        a = idx_all[...].reshape(B, GRID // B, BLK)
        eq = (a[:, None, :, :] == a[None, :, :, :])
        c = jnp.sum(eq.astype(jnp.float32), axis=0)
        s = jnp.sum(jnp.log(c * 0.125 + 1e-10)) * 0.125
        ppl_ref[...] = jnp.full((1, 1), jnp.exp(-s), dtype=jnp.float32)


@functools.partial(jax.jit, static_argnames=("interpret",))
def _vq_call(flat_x, emb, e2, interpret=False):
    out = pl.pallas_call(
        _vq_body,
        grid=(GRID,),
        in_specs=[
            pl.BlockSpec((BLK, D), lambda i: (i, 0)),
            pl.BlockSpec((D, K), lambda i: (0, 0)),
            pl.BlockSpec((1, K), lambda i: (0, 0)),
        ],
        out_specs=[
            pl.BlockSpec((BLK, K), lambda i: (i, 0)),
            pl.BlockSpec((BLK, K), lambda i: (i, 0)),
            pl.BlockSpec((BLK, D), lambda i: (i, 0)),
            pl.BlockSpec((1, 1, BLK), lambda i: (i, 0, 0)),
            pl.BlockSpec((1, 1), lambda i: (0, 0)),
            pl.BlockSpec((1, 1), lambda i: (0, 0)),
        ],
        out_shape=[
            jax.ShapeDtypeStruct((N, K), jnp.float32),
            jax.ShapeDtypeStruct((N, K), jnp.float32),
            jax.ShapeDtypeStruct((N, D), jnp.float32),
            jax.ShapeDtypeStruct((GRID, 1, BLK), jnp.int32),
            jax.ShapeDtypeStruct((1, 1), jnp.float32),
            jax.ShapeDtypeStruct((1, 1), jnp.float32),
        ],
        scratch_shapes=[
            pltpu.VMEM((GRID, BLK), jnp.int32),
            pltpu.SMEM((1, 1), jnp.float32),
        ],
        interpret=interpret,
    )(flat_x, emb, e2)
    return out


def kernel(inputs, embeddings):
    flat_x = inputs.reshape(N, D)
    e2 = jnp.sum(embeddings ** 2, axis=0, keepdims=True)
    dist, enc, q, idx, loss, ppl = _vq_call(flat_x, embeddings, e2)
    quantized_st = q.reshape(inputs.shape)
    encodings = enc.reshape(B, 32, 32, K)
    encoding_indices = idx.reshape(B, 32, 32, 1)
    return (quantized_st, loss.reshape(()), ppl.reshape(()),
            encodings, encoding_indices, dist)


# speedup vs baseline: 2.6920x; 1.8895x over previous
import functools

import jax
import jax.numpy as jnp
from jax import lax
from jax.experimental import pallas as pl
from jax.experimental.pallas import tpu as pltpu
from jax.experimental.pallas import tpu_sc as plsc

D = 256
K = 8192
B = 8
N = B * 1024
BLK = 256
GRID = N // BLK
COMMIT = 0.25

NW = 32
B_PER_W = N // NW
IDX_CHUNK = 128


def _vq_body(x_ref, e_ref, e2_ref, dist_ref, enc_ref, idx_ref,
             loss_ref, ppl_ref, idx_all, loss_acc):
    i = pl.program_id(0)

    xb = x_ref[...]
    mm = jax.lax.dot_general(
        xb, e_ref[...], (((1,), (0,)), ((), ())),
        preferred_element_type=jnp.float32)
    x2 = jnp.sum(xb * xb, axis=1, keepdims=True)
    dist = (x2 - 2.0 * mm) + e2_ref[...]
    dist_ref[...] = dist

    min_d = jnp.min(dist, axis=1, keepdims=True)
    iota = jax.lax.broadcasted_iota(jnp.int32, (BLK, K), 1)
    sel = jnp.where(dist == min_d, iota, jnp.int32(2147483647))
    idx = jnp.min(sel, axis=1).astype(jnp.int32)
    idx_ref[...] = idx[None, None, :]
    idx_all[i, :] = idx

    enc_ref[...] = (iota == idx[:, None]).astype(jnp.float32)

    @pl.when(i == 0)
    def _():
        loss_acc[0, 0] = 0.0

    loss_acc[0, 0] += jnp.sum(min_d)

    @pl.when(i == GRID - 1)
    def _():
        loss_ref[...] = jnp.full(
            (1, 1), loss_acc[0, 0] * ((1.0 + COMMIT) / (N * D)),
            dtype=jnp.float32)
        a = idx_all[...].reshape(B, GRID // B, BLK)
        eq = (a[:, None, :, :] == a[None, :, :, :])
        c = jnp.sum(eq.astype(jnp.float32), axis=0)
        s = jnp.sum(jnp.log(c * 0.125 + 1e-10)) * 0.125
        ppl_ref[...] = jnp.full((1, 1), jnp.exp(-s), dtype=jnp.float32)


@jax.jit
def _vq_tc(flat_x, emb, e2):
    return pl.pallas_call(
        _vq_body,
        grid=(GRID,),
        in_specs=[
            pl.BlockSpec((BLK, D), lambda i: (i, 0)),
            pl.BlockSpec((D, K), lambda i: (0, 0)),
            pl.BlockSpec((1, K), lambda i: (0, 0)),
        ],
        out_specs=[
            pl.BlockSpec((BLK, K), lambda i: (i, 0)),
            pl.BlockSpec((BLK, K), lambda i: (i, 0)),
            pl.BlockSpec((1, 1, BLK), lambda i: (i, 0, 0)),
            pl.BlockSpec((1, 1), lambda i: (0, 0)),
            pl.BlockSpec((1, 1), lambda i: (0, 0)),
        ],
        out_shape=[
            jax.ShapeDtypeStruct((N, K), jnp.float32),
            jax.ShapeDtypeStruct((N, K), jnp.float32),
            jax.ShapeDtypeStruct((GRID, 1, BLK), jnp.int32),
            jax.ShapeDtypeStruct((1, 1), jnp.float32),
            jax.ShapeDtypeStruct((1, 1), jnp.float32),
        ],
        scratch_shapes=[
            pltpu.VMEM((GRID, BLK), jnp.int32),
            pltpu.SMEM((1, 1), jnp.float32),
        ],
    )(flat_x, emb, e2)


def _gather_body(table_hbm, idx_hbm, out_hbm, idx_v, rows_v, sem):
    wid = lax.axis_index("s") * 2 + lax.axis_index("c")
    base = wid * B_PER_W
    pltpu.sync_copy(idx_hbm.at[wid], idx_v)
    for j in range(B_PER_W // IDX_CHUNK):
        pltpu.async_copy(table_hbm.at[idx_v.at[j]],
                         rows_v.at[pl.ds(j * IDX_CHUNK, IDX_CHUNK)],
                         sem).wait()
    pltpu.sync_copy(rows_v, out_hbm.at[pl.ds(base, B_PER_W)])


@jax.jit
def _vq_sc_gather(table, idx3):
    mesh = plsc.VectorSubcoreMesh(core_axis_name="c", subcore_axis_name="s")
    f = functools.partial(
        pl.kernel, mesh=mesh,
        out_type=jax.ShapeDtypeStruct((N, D), jnp.float32),
        scratch_types=[
            pltpu.VMEM((B_PER_W // IDX_CHUNK, IDX_CHUNK), jnp.int32),
            pltpu.VMEM((B_PER_W, D), jnp.float32),
            pltpu.SemaphoreType.DMA,
        ],
    )(_gather_body)
    return f(table, idx3)


def kernel(inputs, embeddings):
    flat_x = inputs.reshape(N, D)
    e2 = jnp.sum(embeddings ** 2, axis=0, keepdims=True)
    dist, enc, idx, loss, ppl = _vq_tc(flat_x, embeddings, e2)
    table = embeddings.T
    idx3 = idx.reshape(NW, B_PER_W // IDX_CHUNK, IDX_CHUNK)
    q = _vq_sc_gather(table, idx3)
    quantized_st = q.reshape(inputs.shape)
    encodings = enc.reshape(B, 32, 32, K)
    encoding_indices = idx.reshape(B, 32, 32, 1)
    return (quantized_st, loss.reshape(()), ppl.reshape(()),
            encodings, encoding_indices, dist)
